# baseline (speedup 1.0000x reference)
import jax
import jax.numpy as jnp
from jax import lax
from jax.experimental import pallas as pl
from jax.experimental.pallas import tpu as pltpu
from jax.experimental.pallas import tpu_sc as plsc

_MIN_PIXELS = 128
_RATIO = 16
_H = 32
_HW = _H * _H
_L = 16
_CHUNK = 128


def _pool_body(mask_ref, sel_ref):
    bs = mask_ref.shape[0]
    r = lax.broadcasted_iota(jnp.int32, (_H, 512), 0)
    c = lax.broadcasted_iota(jnp.int32, (_H, 512), 1)
    pool = (c // _RATIO == r).astype(jnp.float32)
    for b in range(bs):
        m = mask_ref[b].astype(jnp.float32)
        t = jnp.dot(pool, m, preferred_element_type=jnp.float32)
        cnt = lax.dot_general(t, pool, (((1,), (1,)), ((), ())),
                              preferred_element_type=jnp.float32)
        sel_ref[b] = (cnt >= _MIN_PIXELS).astype(jnp.float32)


def _transpose_body(emb_ref, out_ref):
    out_ref[0] = emb_ref[0].T


_NBUF = 2
_BATCH = 16
_GRP = 3


def _sc_mean_body(ids_hbm, sel_hbm, embt_hbm, out_hbm,
                  ids_v, selbatch_v, rowbuf_v, acc_v, seglist_s, sems):
    img = lax.axis_index("s")
    parity = lax.axis_index("c")
    S = ids_v.shape[0]
    C = acc_v.shape[1]
    nvec = C // _L
    nchunk = _HW // _CHUNK

    pltpu.sync_copy(ids_hbm, ids_v)

    def scan_body(g, cnt):
        idv = ids_v[pl.ds(g * _L, _L)]
        for l in range(_L):
            s = g * _L + l
            match = jnp.logical_and(idv[l] == img,
                                    lax.rem(s, 2) == parity)

            @pl.when(match)
            def _(cnt=cnt, s=s):
                seglist_s[cnt] = s

            cnt = cnt + match.astype(jnp.int32)
        return cnt

    nseg = lax.fori_loop(0, S // _L, scan_body, jnp.int32(0))
    nbatch = (nseg + _BATCH - 1) // _BATCH

    def emb_copy(chunk, buf):
        return pltpu.make_async_copy(
            embt_hbm.at[img, pl.ds(chunk * _CHUNK * C, _CHUNK * C)],
            rowbuf_v.at[pl.ds(buf * _CHUNK * C, _CHUNK * C)], sems.at[buf])

    def batch_body(b, _):
        bsize = jnp.minimum(nseg - b * _BATCH, _BATCH)

        def sel_copy(j):
            s = seglist_s[b * _BATCH + j]
            return pltpu.make_async_copy(
                sel_hbm.at[s], selbatch_v.at[pl.ds(j * _HW, _HW)],
                sems.at[_NBUF])

        def ld_start(j, _):
            sel_copy(j).start()
            for v in range(nvec):
                acc_v[j, pl.ds(v * _L, _L)] = jnp.zeros((_L,), jnp.float32)
            return 0

        def ld_wait(j, _):
            sel_copy(j).wait()
            return 0

        lax.fori_loop(0, bsize, ld_start, 0)
        lax.fori_loop(0, bsize, ld_wait, 0)

        emb_copy(0, 0).start()

        def chunk_body(chunk, _):
            buf = lax.rem(chunk, _NBUF)
            emb_copy(chunk, buf).wait()

            @pl.when(chunk + 1 < nchunk)
            def _():
                emb_copy(chunk + 1, lax.rem(chunk + 1, _NBUF)).start()

            def group_body(j3, _):
                j0 = j3 * _GRP

                selbase0 = j0 * _HW + chunk * _CHUNK
                rowbase0 = buf * _CHUNK * C

                def g_body(g, accs):
                    selbase = selbase0 + g * _L
                    rowbase = rowbase0 + g * (_L * C)
                    pvs = [selbatch_v[pl.ds(selbase + t * _HW, _L)]
                           for t in range(_GRP)]
                    accs = list(accs)
                    for l in range(_L):
                        ws = [pvs[t][l] for t in range(_GRP)]
                        for v in range(nvec):
                            rv = rowbuf_v[pl.ds(rowbase + l * C + v * _L, _L)]
                            for t in range(_GRP):
                                k = t * nvec + v
                                accs[k] = accs[k] + rv * ws[t]
                    return tuple(accs)

                accs = tuple(acc_v[j0 + t, pl.ds(v * _L, _L)]
                             for t in range(_GRP) for v in range(nvec))
                accs = plsc.parallel_loop(0, _CHUNK // _L, carry=accs)(g_body)
                for t in range(_GRP):
                    for v in range(nvec):
                        acc_v[j0 + t, pl.ds(v * _L, _L)] = accs[t * nvec + v]
                return 0

            lax.fori_loop(0, (bsize + _GRP - 1) // _GRP, group_body, 0)
            return 0

        lax.fori_loop(0, nchunk, chunk_body, 0)

        def out_copy(j):
            s = seglist_s[b * _BATCH + j]
            return pltpu.make_async_copy(acc_v.at[j], out_hbm.at[s],
                                         sems.at[_NBUF + 1])

        def fin_start(j, _):
            out_copy(j).start()
            return 0

        def fin_wait(j, _):
            out_copy(j).wait()
            return 0

        lax.fori_loop(0, bsize, fin_start, 0)
        lax.fori_loop(0, bsize, fin_wait, 0)
        return 0

    lax.fori_loop(0, nbatch, batch_body, 0)


def _div_body(sel_ref, num_ref, out_ref):
    den = jnp.sum(sel_ref[...], axis=1, keepdims=True)
    out_ref[...] = num_ref[...] / den


def kernel(binary_masks, image_ids, relative_segment_ids, coords,
           sam_encoder_embeddings):
    S = binary_masks.shape[0]
    n_envs = sam_encoder_embeddings.shape[0]
    C = sam_encoder_embeddings.shape[2]
    masks = binary_masks.reshape(S, 512, 512)
    emb = sam_encoder_embeddings.reshape(n_envs, C, _HW)

    BS = 4
    sel = pl.pallas_call(
        _pool_body,
        grid=(S // BS,),
        in_specs=[pl.BlockSpec((BS, 512, 512), lambda i: (i, 0, 0))],
        out_specs=pl.BlockSpec((BS, _H, _H), lambda i: (i, 0, 0)),
        out_shape=jax.ShapeDtypeStruct((S, _H, _H), jnp.float32),
    )(masks)
    sel2 = sel.reshape(S, _HW)

    embt = pl.pallas_call(
        _transpose_body,
        grid=(n_envs,),
        in_specs=[pl.BlockSpec((1, C, _HW), lambda i: (i, 0, 0))],
        out_specs=pl.BlockSpec((1, _HW, C), lambda i: (i, 0, 0)),
        out_shape=jax.ShapeDtypeStruct((n_envs, _HW, C), jnp.float32),
    )(emb)

    mesh = plsc.VectorSubcoreMesh(core_axis_name="c", subcore_axis_name="s")
    num = pl.kernel(
        _sc_mean_body,
        out_type=jax.ShapeDtypeStruct((S, C), jnp.float32),
        mesh=mesh,
        scratch_types=[
            pltpu.VMEM((S,), jnp.int32),
            pltpu.VMEM(((_BATCH + _GRP - 1) * _HW,), jnp.float32),
            pltpu.VMEM((_NBUF * _CHUNK * C,), jnp.float32),
            pltpu.VMEM((_BATCH + _GRP - 1, C), jnp.float32),
            pltpu.SMEM((S,), jnp.int32),
            pltpu.SemaphoreType.DMA((_NBUF + 2,)),
        ],
    )(image_ids, sel2, embt.reshape(n_envs, _HW * C))

    segs = pl.pallas_call(
        _div_body,
        in_specs=[
            pl.BlockSpec((S, _HW), lambda: (0, 0)),
            pl.BlockSpec((S, C), lambda: (0, 0)),
        ],
        out_specs=pl.BlockSpec((S, C), lambda: (0, 0)),
        out_shape=jax.ShapeDtypeStruct((S, C), jnp.float32),
    )(sel2, num)

    is_latent_tokens = jnp.zeros((S,), dtype=bool)
    return (image_ids, relative_segment_ids, is_latent_tokens, segs, coords)


# speedup vs baseline: 1.1393x; 1.1393x over previous
import jax
import jax.numpy as jnp
from jax import lax
from jax.experimental import pallas as pl
from jax.experimental.pallas import tpu as pltpu
from jax.experimental.pallas import tpu_sc as plsc

_MIN_PIXELS = 128
_RATIO = 16
_H = 32
_HW = _H * _H
_L = 16
_CHUNK = 128


def _pool_body(mask_ref, sel_ref):
    bs = mask_ref.shape[0]
    r = lax.broadcasted_iota(jnp.int32, (_H, 512), 0)
    c = lax.broadcasted_iota(jnp.int32, (_H, 512), 1)
    pool = (c // _RATIO == r).astype(jnp.float32)
    for b in range(bs):
        m = mask_ref[b].astype(jnp.float32)
        t = jnp.dot(pool, m, preferred_element_type=jnp.float32)
        cnt = lax.dot_general(t, pool, (((1,), (1,)), ((), ())),
                              preferred_element_type=jnp.float32)
        sel_ref[b] = (cnt >= _MIN_PIXELS).astype(jnp.float32)


def _transpose_body(emb_ref, out_ref):
    out_ref[0] = emb_ref[0].T


_NBUF = 2
_BATCH = 16
_GRP = 3


def _sc_mean_body(ids_hbm, sel_hbm, embt_hbm, out_hbm,
                  ids_v, selbatch_v, rowbuf_v, acc_v, seglist_s, sems):
    img = lax.axis_index("s")
    parity = lax.axis_index("c")
    S = ids_v.shape[0]
    C = acc_v.shape[1]
    nvec = C // _L
    nchunk = _HW // _CHUNK

    pltpu.sync_copy(ids_hbm, ids_v)

    def scan_body(g, cnt):
        idv = ids_v[pl.ds(g * _L, _L)]
        for l in range(_L):
            s = g * _L + l
            match = jnp.logical_and(idv[l] == img,
                                    lax.rem(s, 2) == parity)

            @pl.when(match)
            def _(cnt=cnt, s=s):
                seglist_s[cnt] = s

            cnt = cnt + match.astype(jnp.int32)
        return cnt

    nseg = lax.fori_loop(0, S // _L, scan_body, jnp.int32(0))
    nbatch = (nseg + _BATCH - 1) // _BATCH

    def emb_copy(chunk, buf):
        return pltpu.make_async_copy(
            embt_hbm.at[img, pl.ds(chunk * _CHUNK * C, _CHUNK * C)],
            rowbuf_v.at[pl.ds(buf * _CHUNK * C, _CHUNK * C)], sems.at[buf])

    def batch_body(b, _):
        bsize = jnp.minimum(nseg - b * _BATCH, _BATCH)

        def sel_copy(j):
            s = seglist_s[b * _BATCH + j]
            return pltpu.make_async_copy(
                sel_hbm.at[s], selbatch_v.at[pl.ds(j * _HW, _HW)],
                sems.at[_NBUF])

        def ld_start(j, _):
            sel_copy(j).start()
            for v in range(nvec):
                acc_v[j, pl.ds(v * _L, _L)] = jnp.zeros((_L,), jnp.float32)
            return 0

        def ld_wait(j, _):
            sel_copy(j).wait()
            return 0

        lax.fori_loop(0, bsize, ld_start, 0)
        lax.fori_loop(0, bsize, ld_wait, 0)

        emb_copy(0, 0).start()

        def chunk_body(chunk, _):
            buf = lax.rem(chunk, _NBUF)
            emb_copy(chunk, buf).wait()

            @pl.when(chunk + 1 < nchunk)
            def _():
                emb_copy(chunk + 1, lax.rem(chunk + 1, _NBUF)).start()

            def group_body(j3, _):
                j0 = j3 * _GRP

                selbase0 = j0 * _HW + chunk * _CHUNK
                rowbase0 = buf * _CHUNK * C

                def g_body(g, accs):
                    selbase = selbase0 + g * _L
                    rowbase = rowbase0 + g * (_L * C)
                    pvs = [selbatch_v[pl.ds(selbase + t * _HW, _L)]
                           for t in range(_GRP)]
                    accs = list(accs)
                    for l in range(_L):
                        ws = [pvs[t][l] for t in range(_GRP)]
                        for v in range(nvec):
                            rv = rowbuf_v[pl.ds(rowbase + l * C + v * _L, _L)]
                            for t in range(_GRP):
                                k = t * nvec + v
                                accs[k] = accs[k] + rv * ws[t]
                    return tuple(accs)

                accs = tuple(acc_v[j0 + t, pl.ds(v * _L, _L)]
                             for t in range(_GRP) for v in range(nvec))
                accs = plsc.parallel_loop(0, _CHUNK // _L, carry=accs)(g_body)
                for t in range(_GRP):
                    for v in range(nvec):
                        acc_v[j0 + t, pl.ds(v * _L, _L)] = accs[t * nvec + v]
                return 0

            lax.fori_loop(0, (bsize + _GRP - 1) // _GRP, group_body, 0)
            return 0

        lax.fori_loop(0, nchunk, chunk_body, 0)

        def out_copy(j):
            s = seglist_s[b * _BATCH + j]
            return pltpu.make_async_copy(acc_v.at[j], out_hbm.at[s],
                                         sems.at[_NBUF + 1])

        def fin_start(j, _):
            out_copy(j).start()
            return 0

        def fin_wait(j, _):
            out_copy(j).wait()
            return 0

        lax.fori_loop(0, bsize, fin_start, 0)
        lax.fori_loop(0, bsize, fin_wait, 0)
        return 0

    lax.fori_loop(0, nbatch, batch_body, 0)


def _div_body(sel_ref, num_ref, out_ref):
    den = jnp.sum(sel_ref[...], axis=1, keepdims=True)
    out_ref[...] = num_ref[...] / den


def kernel(binary_masks, image_ids, relative_segment_ids, coords,
           sam_encoder_embeddings):
    S = binary_masks.shape[0]
    n_envs = sam_encoder_embeddings.shape[0]
    C = sam_encoder_embeddings.shape[2]
    masks = binary_masks.reshape(S, 512, 512)
    emb = sam_encoder_embeddings.reshape(n_envs, C, _HW)

    BS = 4
    sel = pl.pallas_call(
        _pool_body,
        grid=(S // BS,),
        in_specs=[pl.BlockSpec((BS, 512, 512), lambda i: (i, 0, 0))],
        out_specs=pl.BlockSpec((BS, _H, _H), lambda i: (i, 0, 0)),
        out_shape=jax.ShapeDtypeStruct((S, _H, _H), jnp.float32),
    )(masks)
    sel2 = sel.reshape(S, _HW)

    embt = jnp.transpose(emb, (0, 2, 1))

    mesh = plsc.VectorSubcoreMesh(core_axis_name="c", subcore_axis_name="s")
    num = pl.kernel(
        _sc_mean_body,
        out_type=jax.ShapeDtypeStruct((S, C), jnp.float32),
        mesh=mesh,
        scratch_types=[
            pltpu.VMEM((S,), jnp.int32),
            pltpu.VMEM(((_BATCH + _GRP - 1) * _HW,), jnp.float32),
            pltpu.VMEM((_NBUF * _CHUNK * C,), jnp.float32),
            pltpu.VMEM((_BATCH + _GRP - 1, C), jnp.float32),
            pltpu.SMEM((S,), jnp.int32),
            pltpu.SemaphoreType.DMA((_NBUF + 2,)),
        ],
    )(image_ids, sel2, embt.reshape(n_envs, _HW * C))

    segs = pl.pallas_call(
        _div_body,
        in_specs=[
            pl.BlockSpec((S, _HW), lambda: (0, 0)),
            pl.BlockSpec((S, C), lambda: (0, 0)),
        ],
        out_specs=pl.BlockSpec((S, C), lambda: (0, 0)),
        out_shape=jax.ShapeDtypeStruct((S, C), jnp.float32),
    )(sel2, num)

    is_latent_tokens = jnp.zeros((S,), dtype=bool)
    return (image_ids, relative_segment_ids, is_latent_tokens, segs, coords)


# speedup vs baseline: 1.2203x; 1.0711x over previous
import jax
import jax.numpy as jnp
from jax import lax
from jax.experimental import pallas as pl
from jax.experimental.pallas import tpu as pltpu
from jax.experimental.pallas import tpu_sc as plsc

_MIN_PIXELS = 128
_RATIO = 16
_H = 32
_HW = _H * _H
_L = 16
_CHUNK = 128


def _pool_body(mask_ref, sel_ref):
    bs = mask_ref.shape[0]
    r = lax.broadcasted_iota(jnp.int32, (_H, 512), 0)
    c = lax.broadcasted_iota(jnp.int32, (_H, 512), 1)
    pool = (c // _RATIO == r).astype(jnp.float32)
    for b in range(bs):
        m = mask_ref[b].astype(jnp.float32)
        t = jnp.dot(pool, m, preferred_element_type=jnp.float32)
        cnt = lax.dot_general(t, pool, (((1,), (1,)), ((), ())),
                              preferred_element_type=jnp.float32)
        sel_ref[b] = (cnt >= _MIN_PIXELS).astype(jnp.float32)


def _transpose_body(emb_ref, out_ref):
    out_ref[0] = emb_ref[0].T


_NBUF = 2
_BATCH = 16
_GRP = 3


def _sc_mean_body(ids_hbm, sel_hbm, embt_hbm, out_hbm,
                  ids_v, selbatch_v, rowbuf_v, acc_v, seglist_s, sems):
    img = lax.axis_index("s")
    parity = lax.axis_index("c")
    S = ids_v.shape[0]
    C = acc_v.shape[1]
    nvec = C // _L
    nchunk = _HW // _CHUNK

    pltpu.sync_copy(ids_hbm, ids_v)

    def scan_body(g, cnt):
        idv = ids_v[pl.ds(g * _L, _L)]
        for l in range(_L):
            s = g * _L + l
            match = jnp.logical_and(idv[l] == img,
                                    lax.rem(s, 2) == parity)

            @pl.when(match)
            def _(cnt=cnt, s=s):
                seglist_s[cnt] = s

            cnt = cnt + match.astype(jnp.int32)
        return cnt

    nseg = lax.fori_loop(0, S // _L, scan_body, jnp.int32(0))
    nbatch = (nseg + _BATCH - 1) // _BATCH

    def emb_copy(chunk, buf):
        return pltpu.make_async_copy(
            embt_hbm.at[img, pl.ds(chunk * _CHUNK * C, _CHUNK * C)],
            rowbuf_v.at[pl.ds(buf * _CHUNK * C, _CHUNK * C)], sems.at[buf])

    def batch_body(b, _):
        bsize = jnp.minimum(nseg - b * _BATCH, _BATCH)

        def sel_copy(j):
            s = seglist_s[b * _BATCH + j]
            return pltpu.make_async_copy(
                sel_hbm.at[s], selbatch_v.at[pl.ds(j * _HW, _HW)],
                sems.at[_NBUF])

        def ld_start(j, _):
            sel_copy(j).start()
            for v in range(nvec):
                acc_v[j, pl.ds(v * _L, _L)] = jnp.zeros((_L,), jnp.float32)
            return 0

        def ld_wait(j, _):
            sel_copy(j).wait()
            return 0

        lax.fori_loop(0, bsize, ld_start, 0)
        lax.fori_loop(0, bsize, ld_wait, 0)

        emb_copy(0, 0).start()

        def chunk_body(chunk, _):
            buf = lax.rem(chunk, _NBUF)
            emb_copy(chunk, buf).wait()

            @pl.when(chunk + 1 < nchunk)
            def _():
                emb_copy(chunk + 1, lax.rem(chunk + 1, _NBUF)).start()

            def group_body(j3, _):
                j0 = j3 * _GRP

                selbase0 = j0 * _HW + chunk * _CHUNK
                rowbase0 = buf * _CHUNK * C

                def g_body(g, accs):
                    selbase = selbase0 + g * _L
                    rowbase = rowbase0 + g * (_L * C)
                    pvs = [selbatch_v[pl.ds(selbase + t * _HW, _L)]
                           for t in range(_GRP)]
                    accs = list(accs)
                    for l in range(_L):
                        ws = [pvs[t][l] for t in range(_GRP)]
                        for v in range(nvec):
                            rv = rowbuf_v[pl.ds(rowbase + l * C + v * _L, _L)]
                            for t in range(_GRP):
                                k = t * nvec + v
                                accs[k] = accs[k] + rv * ws[t]
                    return tuple(accs)

                accs = tuple(acc_v[j0 + t, pl.ds(v * _L, _L)]
                             for t in range(_GRP) for v in range(nvec))
                accs = plsc.parallel_loop(0, _CHUNK // _L, carry=accs)(g_body)
                for t in range(_GRP):
                    for v in range(nvec):
                        acc_v[j0 + t, pl.ds(v * _L, _L)] = accs[t * nvec + v]
                return 0

            lax.fori_loop(0, (bsize + _GRP - 1) // _GRP, group_body, 0)
            return 0

        lax.fori_loop(0, nchunk, chunk_body, 0)

        def out_copy(j):
            s = seglist_s[b * _BATCH + j]
            return pltpu.make_async_copy(acc_v.at[j], out_hbm.at[s],
                                         sems.at[_NBUF + 1])

        def fin_start(j, _):
            out_copy(j).start()
            return 0

        def fin_wait(j, _):
            out_copy(j).wait()
            return 0

        lax.fori_loop(0, bsize, fin_start, 0)
        lax.fori_loop(0, bsize, fin_wait, 0)
        return 0

    lax.fori_loop(0, nbatch, batch_body, 0)


def _div_body(sel_ref, num_ref, out_ref):
    den = jnp.sum(sel_ref[...], axis=1, keepdims=True)
    out_ref[...] = num_ref[...] / den


def kernel(binary_masks, image_ids, relative_segment_ids, coords,
           sam_encoder_embeddings):
    S = binary_masks.shape[0]
    n_envs = sam_encoder_embeddings.shape[0]
    C = sam_encoder_embeddings.shape[2]
    masks = binary_masks.reshape(S, 512, 512)
    emb = sam_encoder_embeddings.reshape(n_envs, C, _HW)

    BS = 8
    sel = pl.pallas_call(
        _pool_body,
        grid=(S // BS,),
        in_specs=[pl.BlockSpec((BS, 512, 512), lambda i: (i, 0, 0))],
        out_specs=pl.BlockSpec((BS, _H, _H), lambda i: (i, 0, 0)),
        out_shape=jax.ShapeDtypeStruct((S, _H, _H), jnp.float32),
    )(masks)
    sel2 = sel.reshape(S, _HW)

    embt = jnp.transpose(emb, (0, 2, 1))

    mesh = plsc.VectorSubcoreMesh(core_axis_name="c", subcore_axis_name="s")
    num = pl.kernel(
        _sc_mean_body,
        out_type=jax.ShapeDtypeStruct((S, C), jnp.float32),
        mesh=mesh,
        scratch_types=[
            pltpu.VMEM((S,), jnp.int32),
            pltpu.VMEM(((_BATCH + _GRP - 1) * _HW,), jnp.float32),
            pltpu.VMEM((_NBUF * _CHUNK * C,), jnp.float32),
            pltpu.VMEM((_BATCH + _GRP - 1, C), jnp.float32),
            pltpu.SMEM((S,), jnp.int32),
            pltpu.SemaphoreType.DMA((_NBUF + 2,)),
        ],
    )(image_ids, sel2, embt.reshape(n_envs, _HW * C))

    segs = pl.pallas_call(
        _div_body,
        in_specs=[
            pl.BlockSpec((S, _HW), lambda: (0, 0)),
            pl.BlockSpec((S, C), lambda: (0, 0)),
        ],
        out_specs=pl.BlockSpec((S, C), lambda: (0, 0)),
        out_shape=jax.ShapeDtypeStruct((S, C), jnp.float32),
    )(sel2, num)

    is_latent_tokens = jnp.zeros((S,), dtype=bool)
    return (image_ids, relative_segment_ids, is_latent_tokens, segs, coords)


# speedup vs baseline: 1.2330x; 1.0104x over previous
import jax
import jax.numpy as jnp
from jax import lax
from jax.experimental import pallas as pl
from jax.experimental.pallas import tpu as pltpu
from jax.experimental.pallas import tpu_sc as plsc

_MIN_PIXELS = 128
_RATIO = 16
_H = 32
_HW = _H * _H
_L = 16
_CHUNK = 128


def _pool_body(mask_ref, sel_ref):
    bs = mask_ref.shape[0]
    r = lax.broadcasted_iota(jnp.int32, (_H, 512), 0)
    c = lax.broadcasted_iota(jnp.int32, (_H, 512), 1)
    pool = (c // _RATIO == r).astype(jnp.float32)
    for b in range(bs):
        m = mask_ref[b].astype(jnp.float32)
        t = jnp.dot(pool, m, preferred_element_type=jnp.float32)
        cnt = lax.dot_general(t, pool, (((1,), (1,)), ((), ())),
                              preferred_element_type=jnp.float32)
        sel_ref[b] = (cnt >= _MIN_PIXELS).astype(jnp.float32)


def _transpose_body(emb_ref, out_ref):
    out_ref[0] = emb_ref[0].T


_NBUF = 2
_BATCH = 16
_GRP = 3


def _sc_mean_body(ids_hbm, sel_hbm, embt_hbm, out_hbm,
                  ids_v, selbatch_v, rowbuf_v, acc_v, seglist_s, sems):
    img = lax.axis_index("s")
    parity = lax.axis_index("c")
    S = ids_v.shape[0]
    C = acc_v.shape[1]
    nvec = C // _L
    nchunk = _HW // _CHUNK

    pltpu.sync_copy(ids_hbm, ids_v)

    def scan_body(g, cnt):
        idv = ids_v[pl.ds(g * _L, _L)]
        for l in range(_L):
            s = g * _L + l
            match = jnp.logical_and(idv[l] == img,
                                    lax.rem(s, 2) == parity)

            @pl.when(match)
            def _(cnt=cnt, s=s):
                seglist_s[cnt] = s

            cnt = cnt + match.astype(jnp.int32)
        return cnt

    nseg = lax.fori_loop(0, S // _L, scan_body, jnp.int32(0))
    nbatch = (nseg + _BATCH - 1) // _BATCH

    def emb_copy(chunk, buf):
        return pltpu.make_async_copy(
            embt_hbm.at[img, pl.ds(chunk * _CHUNK * C, _CHUNK * C)],
            rowbuf_v.at[pl.ds(buf * _CHUNK * C, _CHUNK * C)], sems.at[buf])

    def batch_body(b, _):
        bsize = jnp.minimum(nseg - b * _BATCH, _BATCH)

        def sel_copy(j):
            s = seglist_s[b * _BATCH + j]
            return pltpu.make_async_copy(
                sel_hbm.at[s], selbatch_v.at[pl.ds(j * _HW, _HW)],
                sems.at[_NBUF])

        def ld_start(j, _):
            sel_copy(j).start()
            for v in range(nvec):
                acc_v[j, pl.ds(v * _L, _L)] = jnp.zeros((_L,), jnp.float32)
            return 0

        def ld_wait(j, _):
            sel_copy(j).wait()
            return 0

        lax.fori_loop(0, bsize, ld_start, 0)
        lax.fori_loop(0, bsize, ld_wait, 0)

        emb_copy(0, 0).start()

        def chunk_body(chunk, _):
            buf = lax.rem(chunk, _NBUF)
            emb_copy(chunk, buf).wait()

            @pl.when(chunk + 1 < nchunk)
            def _():
                emb_copy(chunk + 1, lax.rem(chunk + 1, _NBUF)).start()

            def group_body(j3, _):
                j0 = j3 * _GRP

                selbase0 = j0 * _HW + chunk * _CHUNK
                rowbase0 = buf * _CHUNK * C

                def g_body(g, accs):
                    selbase = selbase0 + g * _L
                    rowbase = rowbase0 + g * (_L * C)
                    pvs = [selbatch_v[pl.ds(selbase + t * _HW, _L)]
                           for t in range(_GRP)]
                    accs = list(accs)
                    for l in range(_L):
                        ws = [pvs[t][l] for t in range(_GRP)]
                        for v in range(nvec):
                            rv = rowbuf_v[pl.ds(rowbase + l * C + v * _L, _L)]
                            for t in range(_GRP):
                                k = t * nvec + v
                                accs[k] = accs[k] + rv * ws[t]
                    return tuple(accs)

                accs = tuple(acc_v[j0 + t, pl.ds(v * _L, _L)]
                             for t in range(_GRP) for v in range(nvec))
                accs = plsc.parallel_loop(0, _CHUNK // _L, carry=accs)(g_body)
                for t in range(_GRP):
                    for v in range(nvec):
                        acc_v[j0 + t, pl.ds(v * _L, _L)] = accs[t * nvec + v]
                return 0

            lax.fori_loop(0, (bsize + _GRP - 1) // _GRP, group_body, 0)
            return 0

        lax.fori_loop(0, nchunk, chunk_body, 0)

        def out_copy(j):
            s = seglist_s[b * _BATCH + j]
            return pltpu.make_async_copy(acc_v.at[j], out_hbm.at[s],
                                         sems.at[_NBUF + 1])

        def fin_start(j, _):
            out_copy(j).start()
            return 0

        def fin_wait(j, _):
            out_copy(j).wait()
            return 0

        lax.fori_loop(0, bsize, fin_start, 0)
        lax.fori_loop(0, bsize, fin_wait, 0)
        return 0

    lax.fori_loop(0, nbatch, batch_body, 0)


def _div_body(sel_ref, num_ref, out_ref):
    den = jnp.sum(sel_ref[...], axis=1, keepdims=True)
    out_ref[...] = num_ref[...] / den


def kernel(binary_masks, image_ids, relative_segment_ids, coords,
           sam_encoder_embeddings):
    S = binary_masks.shape[0]
    n_envs = sam_encoder_embeddings.shape[0]
    C = sam_encoder_embeddings.shape[2]
    masks = binary_masks.reshape(S, 512, 512)
    emb = sam_encoder_embeddings.reshape(n_envs, C, _HW)

    BS = 16
    sel = pl.pallas_call(
        _pool_body,
        grid=(S // BS,),
        in_specs=[pl.BlockSpec((BS, 512, 512), lambda i: (i, 0, 0))],
        out_specs=pl.BlockSpec((BS, _H, _H), lambda i: (i, 0, 0)),
        out_shape=jax.ShapeDtypeStruct((S, _H, _H), jnp.float32),
    )(masks)
    sel2 = sel.reshape(S, _HW)

    embt = jnp.transpose(emb, (0, 2, 1))

    mesh = plsc.VectorSubcoreMesh(core_axis_name="c", subcore_axis_name="s")
    num = pl.kernel(
        _sc_mean_body,
        out_type=jax.ShapeDtypeStruct((S, C), jnp.float32),
        mesh=mesh,
        scratch_types=[
            pltpu.VMEM((S,), jnp.int32),
            pltpu.VMEM(((_BATCH + _GRP - 1) * _HW,), jnp.float32),
            pltpu.VMEM((_NBUF * _CHUNK * C,), jnp.float32),
            pltpu.VMEM((_BATCH + _GRP - 1, C), jnp.float32),
            pltpu.SMEM((S,), jnp.int32),
            pltpu.SemaphoreType.DMA((_NBUF + 2,)),
        ],
    )(image_ids, sel2, embt.reshape(n_envs, _HW * C))

    segs = pl.pallas_call(
        _div_body,
        in_specs=[
            pl.BlockSpec((S, _HW), lambda: (0, 0)),
            pl.BlockSpec((S, C), lambda: (0, 0)),
        ],
        out_specs=pl.BlockSpec((S, C), lambda: (0, 0)),
        out_shape=jax.ShapeDtypeStruct((S, C), jnp.float32),
    )(sel2, num)

    is_latent_tokens = jnp.zeros((S,), dtype=bool)
    return (image_ids, relative_segment_ids, is_latent_tokens, segs, coords)


# speedup vs baseline: 1.2543x; 1.0173x over previous
import jax
import jax.numpy as jnp
from jax import lax
from jax.experimental import pallas as pl
from jax.experimental.pallas import tpu as pltpu
from jax.experimental.pallas import tpu_sc as plsc

_MIN_PIXELS = 128
_RATIO = 16
_H = 32
_HW = _H * _H
_L = 16
_CHUNK = 128


def _pool_body(mask_ref, sel_ref):
    bs = mask_ref.shape[0]
    r = lax.broadcasted_iota(jnp.int32, (_H, 512), 0)
    c = lax.broadcasted_iota(jnp.int32, (_H, 512), 1)
    pool = (c // _RATIO == r).astype(jnp.float32)
    for b in range(bs):
        m = mask_ref[b].astype(jnp.float32)
        t = jnp.dot(pool, m, preferred_element_type=jnp.float32)
        cnt = lax.dot_general(t, pool, (((1,), (1,)), ((), ())),
                              preferred_element_type=jnp.float32)
        sel_ref[b] = (cnt >= _MIN_PIXELS).astype(jnp.float32)


def _transpose_body(emb_ref, out_ref):
    out_ref[0] = emb_ref[0].T


_NBUF = 2
_BATCH = 16
_GRP = 3


def _sc_mean_body(ids_hbm, sel_hbm, embt_hbm, out_hbm,
                  ids_v, selbatch_v, rowbuf_v, acc_v, seglist_s, sems):
    img = lax.axis_index("s")
    parity = lax.axis_index("c")
    S = ids_v.shape[0]
    C = acc_v.shape[1]
    nvec = C // _L
    nchunk = _HW // _CHUNK

    pltpu.sync_copy(ids_hbm, ids_v)

    def scan_body(g, cnt):
        idv = ids_v[pl.ds(g * _L, _L)]
        for l in range(_L):
            s = g * _L + l
            match = jnp.logical_and(idv[l] == img,
                                    lax.rem(s, 2) == parity)

            @pl.when(match)
            def _(cnt=cnt, s=s):
                seglist_s[cnt] = s

            cnt = cnt + match.astype(jnp.int32)
        return cnt

    nseg = lax.fori_loop(0, S // _L, scan_body, jnp.int32(0))
    nbatch = (nseg + _BATCH - 1) // _BATCH

    def emb_copy(chunk, buf):
        return pltpu.make_async_copy(
            embt_hbm.at[img, pl.ds(chunk * _CHUNK * C, _CHUNK * C)],
            rowbuf_v.at[pl.ds(buf * _CHUNK * C, _CHUNK * C)], sems.at[buf])

    def batch_body(b, _):
        bsize = jnp.minimum(nseg - b * _BATCH, _BATCH)

        def sel_copy(j):
            s = seglist_s[b * _BATCH + j]
            return pltpu.make_async_copy(
                sel_hbm.at[s], selbatch_v.at[pl.ds(j * _HW, _HW)],
                sems.at[_NBUF])

        def ld_start(j, _):
            sel_copy(j).start()
            for v in range(nvec):
                acc_v[j, pl.ds(v * _L, _L)] = jnp.zeros((_L,), jnp.float32)
            return 0

        def ld_wait(j, _):
            sel_copy(j).wait()
            return 0

        lax.fori_loop(0, bsize, ld_start, 0)
        lax.fori_loop(0, bsize, ld_wait, 0)

        emb_copy(0, 0).start()

        def chunk_body(chunk, _):
            buf = lax.rem(chunk, _NBUF)
            emb_copy(chunk, buf).wait()

            @pl.when(chunk + 1 < nchunk)
            def _():
                emb_copy(chunk + 1, lax.rem(chunk + 1, _NBUF)).start()

            def group_body(j3, _):
                j0 = j3 * _GRP

                selbase0 = j0 * _HW + chunk * _CHUNK
                rowbase0 = buf * _CHUNK * C

                def g_body(g, accs):
                    selbase = selbase0 + g * _L
                    rowbase = rowbase0 + g * (_L * C)
                    pvs = [selbatch_v[pl.ds(selbase + t * _HW, _L)]
                           for t in range(_GRP)]
                    accs = list(accs)
                    for l in range(_L):
                        ws = [pvs[t][l] for t in range(_GRP)]
                        for v in range(nvec):
                            rv = rowbuf_v[pl.ds(rowbase + l * C + v * _L, _L)]
                            for t in range(_GRP):
                                k = t * nvec + v
                                accs[k] = accs[k] + rv * ws[t]
                    return tuple(accs)

                accs = tuple(acc_v[j0 + t, pl.ds(v * _L, _L)]
                             for t in range(_GRP) for v in range(nvec))
                accs = plsc.parallel_loop(0, _CHUNK // _L, carry=accs)(g_body)
                for t in range(_GRP):
                    for v in range(nvec):
                        acc_v[j0 + t, pl.ds(v * _L, _L)] = accs[t * nvec + v]
                return 0

            lax.fori_loop(0, (bsize + _GRP - 1) // _GRP, group_body, 0)
            return 0

        lax.fori_loop(0, nchunk, chunk_body, 0)

        def out_copy(j):
            s = seglist_s[b * _BATCH + j]
            return pltpu.make_async_copy(acc_v.at[j], out_hbm.at[s],
                                         sems.at[_NBUF + 1])

        def fin_start(j, _):
            out_copy(j).start()
            return 0

        def fin_wait(j, _):
            out_copy(j).wait()
            return 0

        lax.fori_loop(0, bsize, fin_start, 0)
        lax.fori_loop(0, bsize, fin_wait, 0)
        return 0

    lax.fori_loop(0, nbatch, batch_body, 0)


def _div_body(sel_ref, num_ref, out_ref):
    den = jnp.sum(sel_ref[...], axis=1, keepdims=True)
    out_ref[...] = num_ref[...] / den


def kernel(binary_masks, image_ids, relative_segment_ids, coords,
           sam_encoder_embeddings):
    S = binary_masks.shape[0]
    n_envs = sam_encoder_embeddings.shape[0]
    C = sam_encoder_embeddings.shape[2]
    masks = binary_masks.reshape(S, 512, 512)
    emb = sam_encoder_embeddings.reshape(n_envs, C, _HW)

    embt = jnp.transpose(emb, (0, 2, 1)).reshape(n_envs, _HW * C)

    BS = 16
    NSPLIT = 2
    Sh = S // NSPLIT
    mesh = plsc.VectorSubcoreMesh(core_axis_name="c", subcore_axis_name="s")
    seg_parts = []
    for h in range(NSPLIT):
        sel = pl.pallas_call(
            _pool_body,
            grid=(Sh // BS,),
            in_specs=[pl.BlockSpec((BS, 512, 512),
                                   lambda i, h=h: (h * (Sh // BS) + i, 0, 0))],
            out_specs=pl.BlockSpec((BS, _H, _H), lambda i: (i, 0, 0)),
            out_shape=jax.ShapeDtypeStruct((Sh, _H, _H), jnp.float32),
        )(masks)
        sel2 = sel.reshape(Sh, _HW)

        num = pl.kernel(
            _sc_mean_body,
            out_type=jax.ShapeDtypeStruct((Sh, C), jnp.float32),
            mesh=mesh,
            scratch_types=[
                pltpu.VMEM((Sh,), jnp.int32),
                pltpu.VMEM(((_BATCH + _GRP - 1) * _HW,), jnp.float32),
                pltpu.VMEM((_NBUF * _CHUNK * C,), jnp.float32),
                pltpu.VMEM((_BATCH + _GRP - 1, C), jnp.float32),
                pltpu.SMEM((Sh,), jnp.int32),
                pltpu.SemaphoreType.DMA((_NBUF + 2,)),
            ],
        )(lax.slice_in_dim(image_ids, h * Sh, (h + 1) * Sh), sel2, embt)

        seg_parts.append(pl.pallas_call(
            _div_body,
            in_specs=[
                pl.BlockSpec((Sh, _HW), lambda: (0, 0)),
                pl.BlockSpec((Sh, C), lambda: (0, 0)),
            ],
            out_specs=pl.BlockSpec((Sh, C), lambda: (0, 0)),
            out_shape=jax.ShapeDtypeStruct((Sh, C), jnp.float32),
        )(sel2, num))

    segs = jnp.concatenate(seg_parts, axis=0)

    is_latent_tokens = jnp.zeros((S,), dtype=bool)
    return (image_ids, relative_segment_ids, is_latent_tokens, segs, coords)


# speedup vs baseline: 1.3767x; 1.0975x over previous
import jax
import jax.numpy as jnp
from jax import lax
from jax.experimental import pallas as pl
from jax.experimental.pallas import tpu as pltpu
from jax.experimental.pallas import tpu_sc as plsc

_MIN_PIXELS = 128
_RATIO = 16
_H = 32
_HW = _H * _H
_L = 16
_CHUNK = 128


def _pool_body(mask_ref, sel_ref):
    bs = mask_ref.shape[0]
    r = lax.broadcasted_iota(jnp.int32, (_H, 512), 0)
    c = lax.broadcasted_iota(jnp.int32, (_H, 512), 1)
    pool = (c // _RATIO == r).astype(jnp.float32)
    for b in range(bs):
        m = mask_ref[b].astype(jnp.float32)
        t = jnp.dot(pool, m, preferred_element_type=jnp.float32)
        cnt = lax.dot_general(t, pool, (((1,), (1,)), ((), ())),
                              preferred_element_type=jnp.float32)
        sel_ref[b] = (cnt >= _MIN_PIXELS).astype(jnp.float32)


def _transpose_body(emb_ref, out_ref):
    out_ref[0] = emb_ref[0].T


_NBUF = 2
_BATCH = 16
_GRP = 3


def _sc_mean_body(ids_hbm, sel_hbm, embt_hbm, out_hbm,
                  ids_v, selbatch_v, rowbuf_v, acc_v, seglist_s, sems):
    img = lax.axis_index("s")
    parity = lax.axis_index("c")
    S = ids_v.shape[0]
    C = acc_v.shape[1]
    nvec = C // _L
    nchunk = _HW // _CHUNK

    pltpu.sync_copy(ids_hbm, ids_v)

    def scan_body(g, carry):
        cnt, imgcnt = carry
        idv = ids_v[pl.ds(g * _L, _L)]
        for l in range(_L):
            s = g * _L + l
            mine = idv[l] == img
            match = jnp.logical_and(mine, lax.rem(imgcnt, 2) == parity)

            @pl.when(match)
            def _(cnt=cnt, s=s):
                seglist_s[cnt] = s

            cnt = cnt + match.astype(jnp.int32)
            imgcnt = imgcnt + mine.astype(jnp.int32)
        return (cnt, imgcnt)

    nseg, _ = lax.fori_loop(0, S // _L, scan_body,
                            (jnp.int32(0), jnp.int32(0)))
    nbatch = (nseg + _BATCH - 1) // _BATCH

    def emb_copy(chunk, buf):
        return pltpu.make_async_copy(
            embt_hbm.at[img, pl.ds(chunk * _CHUNK * C, _CHUNK * C)],
            rowbuf_v.at[pl.ds(buf * _CHUNK * C, _CHUNK * C)], sems.at[buf])

    def batch_body(b, _):
        bsize = jnp.minimum(nseg - b * _BATCH, _BATCH)

        def sel_copy(j):
            s = seglist_s[b * _BATCH + j]
            return pltpu.make_async_copy(
                sel_hbm.at[s], selbatch_v.at[pl.ds(j * _HW, _HW)],
                sems.at[_NBUF])

        def ld_start(j, _):
            sel_copy(j).start()
            for v in range(nvec):
                acc_v[j, pl.ds(v * _L, _L)] = jnp.zeros((_L,), jnp.float32)
            return 0

        def ld_wait(j, _):
            sel_copy(j).wait()
            return 0

        lax.fori_loop(0, bsize, ld_start, 0)
        lax.fori_loop(0, bsize, ld_wait, 0)

        emb_copy(0, 0).start()

        def chunk_body(chunk, _):
            buf = lax.rem(chunk, _NBUF)
            emb_copy(chunk, buf).wait()

            @pl.when(chunk + 1 < nchunk)
            def _():
                emb_copy(chunk + 1, lax.rem(chunk + 1, _NBUF)).start()

            def group_body(j3, _):
                j0 = j3 * _GRP

                selbase0 = j0 * _HW + chunk * _CHUNK
                rowbase0 = buf * _CHUNK * C

                def g_body(g, accs):
                    selbase = selbase0 + g * _L
                    rowbase = rowbase0 + g * (_L * C)
                    pvs = [selbatch_v[pl.ds(selbase + t * _HW, _L)]
                           for t in range(_GRP)]
                    accs = list(accs)
                    for l in range(_L):
                        ws = [pvs[t][l] for t in range(_GRP)]
                        for v in range(nvec):
                            rv = rowbuf_v[pl.ds(rowbase + l * C + v * _L, _L)]
                            for t in range(_GRP):
                                k = t * nvec + v
                                accs[k] = accs[k] + rv * ws[t]
                    return tuple(accs)

                accs = tuple(acc_v[j0 + t, pl.ds(v * _L, _L)]
                             for t in range(_GRP) for v in range(nvec))
                accs = plsc.parallel_loop(0, _CHUNK // _L, carry=accs)(g_body)
                for t in range(_GRP):
                    for v in range(nvec):
                        acc_v[j0 + t, pl.ds(v * _L, _L)] = accs[t * nvec + v]
                return 0

            lax.fori_loop(0, (bsize + _GRP - 1) // _GRP, group_body, 0)
            return 0

        lax.fori_loop(0, nchunk, chunk_body, 0)

        def out_copy(j):
            s = seglist_s[b * _BATCH + j]
            return pltpu.make_async_copy(acc_v.at[j], out_hbm.at[s],
                                         sems.at[_NBUF + 1])

        def fin_start(j, _):
            out_copy(j).start()
            return 0

        def fin_wait(j, _):
            out_copy(j).wait()
            return 0

        lax.fori_loop(0, bsize, fin_start, 0)
        lax.fori_loop(0, bsize, fin_wait, 0)
        return 0

    lax.fori_loop(0, nbatch, batch_body, 0)


def _div_body(sel_ref, num_ref, out_ref):
    den = jnp.sum(sel_ref[...], axis=1, keepdims=True)
    out_ref[...] = num_ref[...] / den


def kernel(binary_masks, image_ids, relative_segment_ids, coords,
           sam_encoder_embeddings):
    S = binary_masks.shape[0]
    n_envs = sam_encoder_embeddings.shape[0]
    C = sam_encoder_embeddings.shape[2]
    masks = binary_masks.reshape(S, 512, 512)
    emb = sam_encoder_embeddings.reshape(n_envs, C, _HW)

    embt = jnp.transpose(emb, (0, 2, 1)).reshape(n_envs, _HW * C)

    BS = 16
    NSPLIT = 2
    Sh = S // NSPLIT
    mesh = plsc.VectorSubcoreMesh(core_axis_name="c", subcore_axis_name="s")
    seg_parts = []
    for h in range(NSPLIT):
        sel = pl.pallas_call(
            _pool_body,
            grid=(Sh // BS,),
            in_specs=[pl.BlockSpec((BS, 512, 512),
                                   lambda i, h=h: (h * (Sh // BS) + i, 0, 0))],
            out_specs=pl.BlockSpec((BS, _H, _H), lambda i: (i, 0, 0)),
            out_shape=jax.ShapeDtypeStruct((Sh, _H, _H), jnp.float32),
        )(masks)
        sel2 = sel.reshape(Sh, _HW)

        num = pl.kernel(
            _sc_mean_body,
            out_type=jax.ShapeDtypeStruct((Sh, C), jnp.float32),
            mesh=mesh,
            scratch_types=[
                pltpu.VMEM((Sh,), jnp.int32),
                pltpu.VMEM(((_BATCH + _GRP - 1) * _HW,), jnp.float32),
                pltpu.VMEM((_NBUF * _CHUNK * C,), jnp.float32),
                pltpu.VMEM((_BATCH + _GRP - 1, C), jnp.float32),
                pltpu.SMEM((Sh,), jnp.int32),
                pltpu.SemaphoreType.DMA((_NBUF + 2,)),
            ],
        )(lax.slice_in_dim(image_ids, h * Sh, (h + 1) * Sh), sel2, embt)

        seg_parts.append(pl.pallas_call(
            _div_body,
            in_specs=[
                pl.BlockSpec((Sh, _HW), lambda: (0, 0)),
                pl.BlockSpec((Sh, C), lambda: (0, 0)),
            ],
            out_specs=pl.BlockSpec((Sh, C), lambda: (0, 0)),
            out_shape=jax.ShapeDtypeStruct((Sh, C), jnp.float32),
        )(sel2, num))

    segs = jnp.concatenate(seg_parts, axis=0)

    is_latent_tokens = jnp.zeros((S,), dtype=bool)
    return (image_ids, relative_segment_ids, is_latent_tokens, segs, coords)
